# baseline (speedup 1.0000x reference)
from itertools import combinations

import numpy as np
import jax
import jax.numpy as jnp
from jax.experimental import pallas as pl
from jax.experimental.pallas import tpu as pltpu

_D = 1024
_H = 16
_DH = 64
_P = 4
_PD = 6
_NG = 21
_NGP = 24
_DECAY = 0.99
_C = 1024
_KC = 512
_RC = 512
_G = 16
_SCALE = _DH ** -0.5
_LN_DECAY = float(np.log(_DECAY))
_DECAY_C = float(_DECAY ** _C)

_PAIRS = list(combinations(range(_P), 2))
_TI, _TJ = np.triu_indices(_PD)


def _np_consts():
    ea = np.zeros((8, 8), np.float32)
    eb = np.zeros((8, 8), np.float32)
    ec = np.zeros((8, 8), np.float32)
    ed = np.zeros((8, 8), np.float32)
    for kk, (i, j) in enumerate(_PAIRS):
        ea[kk, i] = 1.0
        eb[kk, 4 + j] = 1.0
        ec[kk, j] = 1.0
        ed[kk, 4 + i] = 1.0
    eti = np.zeros((_NGP, 8), np.float32)
    etj = np.zeros((_NGP, 8), np.float32)
    for kk in range(_NG):
        eti[kk, _TI[kk]] = 1.0
        etj[kk, _TJ[kk]] = 1.0
    eye = np.eye(_G, dtype=np.float32)
    ea16 = np.kron(eye, ea)
    eb16 = np.kron(eye, eb)
    ec16 = np.kron(eye, ec)
    ed16 = np.kron(eye, ed)
    seg16 = np.kron(eye, np.ones((8, 8), np.float32))
    eti16 = np.kron(eye, eti)
    etj16 = np.kron(eye, etj)
    i = np.arange(_C)[:, None]
    s = np.arange(_C)[None, :]
    lmat = np.where(s < i, _DECAY ** np.maximum(i - 1 - s, 0), 0.0).astype(np.float32)
    lt = np.ascontiguousarray(lmat.T)
    dvec = (_DECAY ** (_C - 1 - np.arange(_C))).astype(np.float32).reshape(1, _C)
    return ea16, eb16, ec16, ed16, seg16, eti16, etj16, lt, dvec


_EA, _EB, _EC, _ED, _SEG, _ETI, _ETJ, _LT, _DVEC = _np_consts()


def _proj_kernel(x_ref, qw_ref, qb_ref, w1_ref, w2_ref, gw_ref, gb_ref,
                 qkv_ref, p1_ref, p2_ref, gl_ref):
    xb = x_ref[...]
    cdims = (((0,), (0,)), ((), ()))
    qkv = jax.lax.dot_general(qw_ref[...], xb, cdims,
                              preferred_element_type=jnp.float32) + qb_ref[...]
    qkv_ref[...] = qkv.astype(jnp.bfloat16)
    p1_ref[...] = jax.lax.dot_general(w1_ref[...], xb, cdims,
                                      preferred_element_type=jnp.float32)
    p2_ref[...] = jax.lax.dot_general(w2_ref[...], xb, cdims,
                                      preferred_element_type=jnp.float32)
    gl_ref[...] = jax.lax.dot_general(gw_ref[...], xb, cdims,
                                      preferred_element_type=jnp.float32) + gb_ref[...]


def _attn_gram_kernel(q_ref, k_ref, v_ref, pw_ref, gl_ref, lt_ref, dv_ref,
                      ea_ref, eb_ref, ec_ref, ed_ref, seg_ref, eti_ref, etj_ref,
                      m1_ref, m1b_ref, m2_ref, m2b_ref,
                      out_ref, s_ref):
    qc = pl.program_id(1)
    t0 = qc * _C
    f32 = jnp.float32
    bf16 = jnp.bfloat16

    @pl.when(qc == 0)
    def _():
        s_ref[...] = jnp.zeros((_G * _NGP, 1), f32)

    cdA = (((0,), (0,)), ((), ()))
    cdS = (((1,), (0,)), ((), ()))
    ones_row = jnp.ones((8, _KC), bf16)

    def one_head_chunk(g, off, m, acc, masked, moff=0):
        kc = k_ref[g, :, pl.ds(off, _KC)]
        st = jax.lax.dot_general(kc, q_ref[g], cdA,
                                 preferred_element_type=f32) * _SCALE
        if masked:
            ki = jax.lax.broadcasted_iota(jnp.int32, (_KC, _C), 0)
            qi = jax.lax.broadcasted_iota(jnp.int32, (_KC, _C), 1)
            st = jnp.where(ki + moff > qi, -1e30, st)
        m_new = jnp.maximum(m, jnp.max(st, axis=0, keepdims=True))
        alpha = jnp.exp(m - m_new)
        p = jnp.exp(st - m_new)
        va = jnp.concatenate([v_ref[g, :, pl.ds(off, _KC)], ones_row], axis=0)
        acc_new = acc * alpha + jax.lax.dot_general(
            va, p.astype(bf16), cdS, preferred_element_type=f32)
        return m_new, acc_new

    def body(j, carry):
        off = pl.multiple_of(j * _KC, _KC)
        return tuple(one_head_chunk(g, off, *carry[g], masked=False)
                     for g in range(_G))

    init = tuple((jnp.full((1, _C), -1e30, f32),
                  jnp.zeros((_DH + 8, _C), f32)) for _ in range(_G))
    carry = jax.lax.fori_loop(0, (_C // _KC) * qc, body, init)
    seqs = []
    for g in range(_G):
        m, acc = carry[g]
        for dj in range(_C // _KC):
            m, acc = one_head_chunk(g, t0 + dj * _KC, m, acc,
                                    masked=True, moff=dj * _KC)
        seqs.append(acc[0:_DH] / acc[_DH:_DH + 1])

    pwa = pw_ref[...]
    a = jnp.dot(ea_ref[...], pwa, preferred_element_type=f32)
    b = jnp.dot(eb_ref[...], pwa, preferred_element_type=f32)
    c = jnp.dot(ec_ref[...], pwa, preferred_element_type=f32)
    d = jnp.dot(ed_ref[...], pwa, preferred_element_type=f32)
    parts = a * b - c * d
    s2 = jnp.dot(seg_ref[...], (parts * parts).astype(bf16),
                 preferred_element_type=f32)
    nr = jnp.maximum(jnp.sqrt(s2), 1e-12)
    wl = (parts / nr).astype(bf16)
    u = jnp.dot(eti_ref[...], wl, preferred_element_type=f32)
    v = jnp.dot(etj_ref[...], wl, preferred_element_type=f32)
    o = u * v

    carry_s = s_ref[...]
    dp = jnp.exp(jax.lax.broadcasted_iota(jnp.int32, (_G * _NGP, _C), 1).astype(f32)
                 * _LN_DECAY)
    gf = dp * carry_s + jnp.dot(o.astype(bf16), lt_ref[...],
                                preferred_element_type=f32)
    s_ref[...] = _DECAY_C * carry_s + jnp.sum(o * dv_ref[...], axis=1, keepdims=True)

    pre = jnp.dot(m1_ref[...], gf.astype(bf16),
                  preferred_element_type=f32) + m1b_ref[...]
    h1 = 0.5 * pre * (1.0 + jax.lax.erf(pre * 0.7071067811865476))
    mem = jnp.dot(m2_ref[...], h1.astype(bf16),
                  preferred_element_type=f32) + m2b_ref[...]

    for g in range(_G):
        gate = jax.nn.sigmoid(gl_ref[0, g:g + 1, :])
        out_ref[g] = (seqs[g] + gate * mem[g * _DH:(g + 1) * _DH]).astype(bf16)


def _out_kernel(c_ref, w_ref, b_ref, o_ref):
    o_ref[...] = jax.lax.dot_general(
        c_ref[...], w_ref[...], (((0,), (0,)), ((), ())),
        preferred_element_type=jnp.float32) + b_ref[...]


def kernel(x, qkv_w, qkv_b, w1_w, w2_w, mlp1_w, mlp1_b, mlp2_w, mlp2_b,
           gate_w, gate_b, out_w, out_b):
    bsz, t, dm = x.shape
    f32 = jnp.float32
    bf16 = jnp.bfloat16
    rows = bsz * t
    ngrid = rows // _RC
    nq = t // _C
    hh = _H

    xt = jnp.transpose(x.reshape(rows, dm).astype(bf16))

    qkvt, p1t, p2t, glt = pl.pallas_call(
        _proj_kernel,
        grid=(ngrid,),
        in_specs=[
            pl.BlockSpec((dm, _RC), lambda i: (0, i)),
            pl.BlockSpec((dm, 3 * dm), lambda i: (0, 0)),
            pl.BlockSpec((3 * dm, 1), lambda i: (0, 0)),
            pl.BlockSpec((dm, _H * _P), lambda i: (0, 0)),
            pl.BlockSpec((dm, _H * _P), lambda i: (0, 0)),
            pl.BlockSpec((dm, _H), lambda i: (0, 0)),
            pl.BlockSpec((_H, 1), lambda i: (0, 0)),
        ],
        out_specs=[
            pl.BlockSpec((3 * dm, _RC), lambda i: (0, i)),
            pl.BlockSpec((_H * _P, _RC), lambda i: (0, i)),
            pl.BlockSpec((_H * _P, _RC), lambda i: (0, i)),
            pl.BlockSpec((_H, _RC), lambda i: (0, i)),
        ],
        out_shape=[
            jax.ShapeDtypeStruct((3 * dm, rows), bf16),
            jax.ShapeDtypeStruct((_H * _P, rows), f32),
            jax.ShapeDtypeStruct((_H * _P, rows), f32),
            jax.ShapeDtypeStruct((_H, rows), f32),
        ],
        compiler_params=pltpu.CompilerParams(
            dimension_semantics=("parallel",),
        ),
    )(xt, qkv_w.astype(bf16), qkv_b.reshape(-1, 1), w1_w.astype(bf16),
      w2_w.astype(bf16), gate_w.astype(bf16), gate_b.reshape(-1, 1))

    qkvh = qkvt.reshape(3 * _H, _DH, rows)
    p1b = p1t.reshape(_H, _P, bsz, t)
    p1s = jnp.concatenate([jnp.zeros((_H, _P, bsz, 1), f32), p1b[..., :-1]], axis=3)
    p2b = p2t.reshape(_H, _P, bsz, t)
    pwt = jnp.concatenate([p1s, p2b], axis=1).reshape(_H, 8, rows)

    m1tp = jnp.concatenate([mlp1_w.T, jnp.zeros((_DH, _NGP - _NG), f32)], axis=1)

    hgn = _H // _G
    combined_t = pl.pallas_call(
        _attn_gram_kernel,
        grid=(bsz * hgn, nq),
        in_specs=[
            pl.BlockSpec((_G, _DH, _C), lambda bh, qc: (bh % hgn, 0, (bh // hgn) * nq + qc)),
            pl.BlockSpec((_G, _DH, t), lambda bh, qc: (hgn + bh % hgn, 0, bh // hgn)),
            pl.BlockSpec((_G, _DH, t), lambda bh, qc: (2 * hgn + bh % hgn, 0, bh // hgn)),
            pl.BlockSpec((_G * 8, _C), lambda bh, qc: (bh % hgn, (bh // hgn) * nq + qc)),
            pl.BlockSpec((1, _G, _C), lambda bh, qc: (bh % hgn, 0, (bh // hgn) * nq + qc)),
            pl.BlockSpec((_C, _C), lambda bh, qc: (0, 0)),
            pl.BlockSpec((1, _C), lambda bh, qc: (0, 0)),
            pl.BlockSpec((_G * 8, _G * 8), lambda bh, qc: (0, 0)),
            pl.BlockSpec((_G * 8, _G * 8), lambda bh, qc: (0, 0)),
            pl.BlockSpec((_G * 8, _G * 8), lambda bh, qc: (0, 0)),
            pl.BlockSpec((_G * 8, _G * 8), lambda bh, qc: (0, 0)),
            pl.BlockSpec((_G * 8, _G * 8), lambda bh, qc: (0, 0)),
            pl.BlockSpec((_G * _NGP, _G * 8), lambda bh, qc: (0, 0)),
            pl.BlockSpec((_G * _NGP, _G * 8), lambda bh, qc: (0, 0)),
            pl.BlockSpec((_G * _DH, _G * _NGP), lambda bh, qc: (0, 0)),
            pl.BlockSpec((_G * _DH, 1), lambda bh, qc: (0, 0)),
            pl.BlockSpec((_G * _DH, _G * _DH), lambda bh, qc: (0, 0)),
            pl.BlockSpec((_G * _DH, 1), lambda bh, qc: (0, 0)),
        ],
        out_specs=pl.BlockSpec((_G, _DH, _C), lambda bh, qc: (bh % hgn, 0, (bh // hgn) * nq + qc)),
        out_shape=jax.ShapeDtypeStruct((_H, _DH, rows), bf16),
        scratch_shapes=[pltpu.VMEM((_G * _NGP, 1), f32)],
        compiler_params=pltpu.CompilerParams(
            dimension_semantics=("parallel", "arbitrary"),
        ),
    )(qkvh, qkvh, qkvh, pwt.reshape(_H * 8, rows).astype(bf16),
      glt.reshape(hgn, _G, rows),
      jnp.asarray(_LT).astype(bf16), jnp.asarray(_DVEC),
      jnp.asarray(_EA).astype(bf16), jnp.asarray(_EB).astype(bf16),
      jnp.asarray(_EC).astype(bf16), jnp.asarray(_ED).astype(bf16),
      jnp.asarray(_SEG).astype(bf16),
      jnp.asarray(_ETI).astype(bf16), jnp.asarray(_ETJ).astype(bf16),
      jnp.kron(jnp.eye(_G, dtype=f32), m1tp).astype(bf16),
      jnp.tile(mlp1_b.reshape(-1, 1), (_G, 1)),
      jnp.kron(jnp.eye(_G, dtype=f32), mlp2_w.T).astype(bf16),
      jnp.tile(mlp2_b.reshape(-1, 1), (_G, 1)))

    out = pl.pallas_call(
        _out_kernel,
        grid=(ngrid,),
        in_specs=[
            pl.BlockSpec((dm, _RC), lambda i: (0, i)),
            pl.BlockSpec((dm, dm), lambda i: (0, 0)),
            pl.BlockSpec((1, dm), lambda i: (0, 0)),
        ],
        out_specs=pl.BlockSpec((_RC, dm), lambda i: (i, 0)),
        out_shape=jax.ShapeDtypeStruct((rows, dm), f32),
        compiler_params=pltpu.CompilerParams(
            dimension_semantics=("parallel",),
        ),
    )(combined_t.reshape(dm, rows), out_w.astype(bf16), out_b.reshape(1, -1))

    return out.reshape(bsz, t, dm)


# speedup vs baseline: 1.2294x; 1.2294x over previous
from itertools import combinations

import numpy as np
import jax
import jax.numpy as jnp
from jax.experimental import pallas as pl
from jax.experimental.pallas import tpu as pltpu

_D = 1024
_H = 16
_DH = 64
_P = 4
_PD = 6
_NG = 21
_NGP = 24
_DECAY = 0.99
_C = 512
_KC = 512
_RC = 512
_G = 16
_SCALE = _DH ** -0.5
_LN_DECAY = float(np.log(_DECAY))
_DECAY_C = float(_DECAY ** _C)

_PAIRS = list(combinations(range(_P), 2))
_TI, _TJ = np.triu_indices(_PD)


def _np_consts():
    ea = np.zeros((8, 8), np.float32)
    eb = np.zeros((8, 8), np.float32)
    ec = np.zeros((8, 8), np.float32)
    ed = np.zeros((8, 8), np.float32)
    for kk, (i, j) in enumerate(_PAIRS):
        ea[kk, i] = 1.0
        eb[kk, 4 + j] = 1.0
        ec[kk, j] = 1.0
        ed[kk, 4 + i] = 1.0
    eti = np.zeros((_NGP, 8), np.float32)
    etj = np.zeros((_NGP, 8), np.float32)
    for kk in range(_NG):
        eti[kk, _TI[kk]] = 1.0
        etj[kk, _TJ[kk]] = 1.0
    eye = np.eye(_G, dtype=np.float32)
    ea16 = np.kron(eye, ea)
    eb16 = np.kron(eye, eb)
    ec16 = np.kron(eye, ec)
    ed16 = np.kron(eye, ed)
    seg16 = np.kron(eye, np.ones((8, 8), np.float32))
    eti16 = np.kron(eye, eti)
    etj16 = np.kron(eye, etj)
    i = np.arange(_C)[:, None]
    s = np.arange(_C)[None, :]
    lmat = np.where(s < i, _DECAY ** np.maximum(i - 1 - s, 0), 0.0).astype(np.float32)
    lt = np.ascontiguousarray(lmat.T)
    dvec = (_DECAY ** (_C - 1 - np.arange(_C))).astype(np.float32).reshape(1, _C)
    dpow = (_DECAY ** np.arange(_C)).astype(np.float32).reshape(1, _C)
    return ea16, eb16, ec16, ed16, seg16, eti16, etj16, lt, dvec, dpow


_EA, _EB, _EC, _ED, _SEG, _ETI, _ETJ, _LT, _DVEC, _DPOW = _np_consts()


def _proj_kernel(x_ref, qw_ref, qb_ref, w1_ref, w2_ref, gw_ref, gb_ref,
                 qkv_ref, p1_ref, p2_ref, gl_ref):
    xb = x_ref[...]
    cdims = (((0,), (0,)), ((), ()))
    qkv = jax.lax.dot_general(qw_ref[...], xb, cdims,
                              preferred_element_type=jnp.float32) + qb_ref[...]
    qkv_ref[...] = qkv.astype(jnp.bfloat16)
    p1_ref[...] = jax.lax.dot_general(w1_ref[...], xb, cdims,
                                      preferred_element_type=jnp.float32)
    p2_ref[...] = jax.lax.dot_general(w2_ref[...], xb, cdims,
                                      preferred_element_type=jnp.float32)
    gl_ref[...] = jax.lax.dot_general(gw_ref[...], xb, cdims,
                                      preferred_element_type=jnp.float32) + gb_ref[...]


def _attn_gram_kernel(q_ref, k_ref, v_ref, pw_ref, gl_ref, lt_ref, dv_ref,
                      dp_ref, ea_ref, eb_ref, ec_ref, ed_ref, seg_ref,
                      eti_ref, etj_ref, m1_ref, m1b_ref, m2_ref, m2b_ref,
                      out_ref, s_ref):
    qc = pl.program_id(1)
    t0 = qc * _C
    f32 = jnp.float32
    bf16 = jnp.bfloat16

    @pl.when(qc == 0)
    def _():
        s_ref[...] = jnp.zeros((_G * _NGP, 1), f32)

    cdA = (((0,), (0,)), ((), ()))
    cdS = (((1,), (0,)), ((), ()))
    ones_row = jnp.ones((8, _KC), bf16)

    ki = jax.lax.broadcasted_iota(jnp.int32, (_KC, _C), 0)
    qi = jax.lax.broadcasted_iota(jnp.int32, (_KC, _C), 1)
    diag_masks = [ki + moff > qi for moff in range(0, _C, _KC)]

    def one_head_chunk(g, off, m, acc, masked, mi=0):
        kc = k_ref[g, :, pl.ds(off, _KC)]
        st = jax.lax.dot_general(kc, q_ref[g], cdA,
                                 preferred_element_type=f32) * _SCALE
        if masked:
            st = jnp.where(diag_masks[mi], -1e30, st)
        m_new = jnp.maximum(m, jnp.max(st, axis=0, keepdims=True))
        alpha = jnp.exp(m - m_new)
        p = jnp.exp(st - m_new)
        va = jnp.concatenate([v_ref[g, :, pl.ds(off, _KC)], ones_row], axis=0)
        acc_new = acc * alpha + jax.lax.dot_general(
            va, p.astype(bf16), cdS, preferred_element_type=f32)
        return m_new, acc_new

    def body(j, carry):
        off = pl.multiple_of(j * _KC, _KC)
        return tuple(one_head_chunk(g, off, *carry[g], masked=False)
                     for g in range(_G))

    init = tuple((jnp.full((1, _C), -1e30, f32),
                  jnp.zeros((_DH + 8, _C), f32)) for _ in range(_G))
    carry = jax.lax.fori_loop(0, (_C // _KC) * qc, body, init)
    seqs = []
    for g in range(_G):
        m, acc = carry[g]
        for dj in range(_C // _KC):
            m, acc = one_head_chunk(g, t0 + dj * _KC, m, acc,
                                    masked=True, mi=dj)
        seqs.append(acc[0:_DH] / acc[_DH:_DH + 1])

    pwa = pw_ref[...]
    a = jnp.dot(ea_ref[...], pwa, preferred_element_type=f32)
    b = jnp.dot(eb_ref[...], pwa, preferred_element_type=f32)
    c = jnp.dot(ec_ref[...], pwa, preferred_element_type=f32)
    d = jnp.dot(ed_ref[...], pwa, preferred_element_type=f32)
    parts = a * b - c * d
    s2 = jnp.dot(seg_ref[...], (parts * parts).astype(bf16),
                 preferred_element_type=f32)
    nr = jnp.maximum(jnp.sqrt(s2), 1e-12)
    wl = (parts / nr).astype(bf16)
    u = jnp.dot(eti_ref[...], wl, preferred_element_type=f32)
    v = jnp.dot(etj_ref[...], wl, preferred_element_type=f32)
    o = u * v

    carry_s = s_ref[...]
    gf = dp_ref[...] * carry_s + jnp.dot(o.astype(bf16), lt_ref[...],
                                         preferred_element_type=f32)
    s_ref[...] = _DECAY_C * carry_s + jnp.sum(o * dv_ref[...], axis=1, keepdims=True)

    pre = jnp.dot(m1_ref[...], gf.astype(bf16),
                  preferred_element_type=f32) + m1b_ref[...]
    h1 = 0.5 * pre * (1.0 + jax.lax.erf(pre * 0.7071067811865476))
    mem = jnp.dot(m2_ref[...], h1.astype(bf16),
                  preferred_element_type=f32) + m2b_ref[...]

    for g in range(_G):
        gate = jax.nn.sigmoid(gl_ref[0, g:g + 1, :])
        out_ref[g] = (seqs[g] + gate * mem[g * _DH:(g + 1) * _DH]).astype(bf16)


def _out_kernel(c_ref, w_ref, b_ref, o_ref):
    o_ref[...] = jax.lax.dot_general(
        c_ref[...], w_ref[...], (((0,), (0,)), ((), ())),
        preferred_element_type=jnp.float32) + b_ref[...]


def kernel(x, qkv_w, qkv_b, w1_w, w2_w, mlp1_w, mlp1_b, mlp2_w, mlp2_b,
           gate_w, gate_b, out_w, out_b):
    bsz, t, dm = x.shape
    f32 = jnp.float32
    bf16 = jnp.bfloat16
    rows = bsz * t
    ngrid = rows // _RC
    nq = t // _C
    hh = _H

    xt = jnp.transpose(x.reshape(rows, dm).astype(bf16))

    qkvt, p1t, p2t, glt = pl.pallas_call(
        _proj_kernel,
        grid=(ngrid,),
        in_specs=[
            pl.BlockSpec((dm, _RC), lambda i: (0, i)),
            pl.BlockSpec((dm, 3 * dm), lambda i: (0, 0)),
            pl.BlockSpec((3 * dm, 1), lambda i: (0, 0)),
            pl.BlockSpec((dm, _H * _P), lambda i: (0, 0)),
            pl.BlockSpec((dm, _H * _P), lambda i: (0, 0)),
            pl.BlockSpec((dm, _H), lambda i: (0, 0)),
            pl.BlockSpec((_H, 1), lambda i: (0, 0)),
        ],
        out_specs=[
            pl.BlockSpec((3 * dm, _RC), lambda i: (0, i)),
            pl.BlockSpec((_H * _P, _RC), lambda i: (0, i)),
            pl.BlockSpec((_H * _P, _RC), lambda i: (0, i)),
            pl.BlockSpec((_H, _RC), lambda i: (0, i)),
        ],
        out_shape=[
            jax.ShapeDtypeStruct((3 * dm, rows), bf16),
            jax.ShapeDtypeStruct((_H * _P, rows), f32),
            jax.ShapeDtypeStruct((_H * _P, rows), f32),
            jax.ShapeDtypeStruct((_H, rows), f32),
        ],
        compiler_params=pltpu.CompilerParams(
            dimension_semantics=("parallel",),
        ),
    )(xt, qkv_w.astype(bf16), qkv_b.reshape(-1, 1), w1_w.astype(bf16),
      w2_w.astype(bf16), gate_w.astype(bf16), gate_b.reshape(-1, 1))

    qkvh = qkvt.reshape(3 * _H, _DH, rows)
    p1b = p1t.reshape(_H, _P, bsz, t)
    p1s = jnp.concatenate([jnp.zeros((_H, _P, bsz, 1), f32), p1b[..., :-1]], axis=3)
    p2b = p2t.reshape(_H, _P, bsz, t)
    pwt = jnp.concatenate([p1s, p2b], axis=1).reshape(_H, 8, rows)

    m1tp = jnp.concatenate([mlp1_w.T, jnp.zeros((_DH, _NGP - _NG), f32)], axis=1)

    hgn = _H // _G
    combined_t = pl.pallas_call(
        _attn_gram_kernel,
        grid=(bsz * hgn, nq),
        in_specs=[
            pl.BlockSpec((_G, _DH, _C), lambda bh, qc: (bh % hgn, 0, (bh // hgn) * nq + qc)),
            pl.BlockSpec((_G, _DH, t), lambda bh, qc: (hgn + bh % hgn, 0, bh // hgn)),
            pl.BlockSpec((_G, _DH, t), lambda bh, qc: (2 * hgn + bh % hgn, 0, bh // hgn)),
            pl.BlockSpec((_G * 8, _C), lambda bh, qc: (bh % hgn, (bh // hgn) * nq + qc)),
            pl.BlockSpec((1, _G, _C), lambda bh, qc: (bh % hgn, 0, (bh // hgn) * nq + qc)),
            pl.BlockSpec((_C, _C), lambda bh, qc: (0, 0)),
            pl.BlockSpec((1, _C), lambda bh, qc: (0, 0)),
            pl.BlockSpec((1, _C), lambda bh, qc: (0, 0)),
            pl.BlockSpec((_G * 8, _G * 8), lambda bh, qc: (0, 0)),
            pl.BlockSpec((_G * 8, _G * 8), lambda bh, qc: (0, 0)),
            pl.BlockSpec((_G * 8, _G * 8), lambda bh, qc: (0, 0)),
            pl.BlockSpec((_G * 8, _G * 8), lambda bh, qc: (0, 0)),
            pl.BlockSpec((_G * 8, _G * 8), lambda bh, qc: (0, 0)),
            pl.BlockSpec((_G * _NGP, _G * 8), lambda bh, qc: (0, 0)),
            pl.BlockSpec((_G * _NGP, _G * 8), lambda bh, qc: (0, 0)),
            pl.BlockSpec((_G * _DH, _G * _NGP), lambda bh, qc: (0, 0)),
            pl.BlockSpec((_G * _DH, 1), lambda bh, qc: (0, 0)),
            pl.BlockSpec((_G * _DH, _G * _DH), lambda bh, qc: (0, 0)),
            pl.BlockSpec((_G * _DH, 1), lambda bh, qc: (0, 0)),
        ],
        out_specs=pl.BlockSpec((_G, _DH, _C), lambda bh, qc: (bh % hgn, 0, (bh // hgn) * nq + qc)),
        out_shape=jax.ShapeDtypeStruct((_H, _DH, rows), bf16),
        scratch_shapes=[pltpu.VMEM((_G * _NGP, 1), f32)],
        compiler_params=pltpu.CompilerParams(
            dimension_semantics=("parallel", "arbitrary"),
        ),
    )(qkvh, qkvh, qkvh, pwt.reshape(_H * 8, rows).astype(bf16),
      glt.reshape(hgn, _G, rows),
      jnp.asarray(_LT).astype(bf16), jnp.asarray(_DVEC), jnp.asarray(_DPOW),
      jnp.asarray(_EA).astype(bf16), jnp.asarray(_EB).astype(bf16),
      jnp.asarray(_EC).astype(bf16), jnp.asarray(_ED).astype(bf16),
      jnp.asarray(_SEG).astype(bf16),
      jnp.asarray(_ETI).astype(bf16), jnp.asarray(_ETJ).astype(bf16),
      jnp.kron(jnp.eye(_G, dtype=f32), m1tp).astype(bf16),
      jnp.tile(mlp1_b.reshape(-1, 1), (_G, 1)),
      jnp.kron(jnp.eye(_G, dtype=f32), mlp2_w.T).astype(bf16),
      jnp.tile(mlp2_b.reshape(-1, 1), (_G, 1)))

    out = pl.pallas_call(
        _out_kernel,
        grid=(ngrid,),
        in_specs=[
            pl.BlockSpec((dm, _RC), lambda i: (0, i)),
            pl.BlockSpec((dm, dm), lambda i: (0, 0)),
            pl.BlockSpec((1, dm), lambda i: (0, 0)),
        ],
        out_specs=pl.BlockSpec((_RC, dm), lambda i: (i, 0)),
        out_shape=jax.ShapeDtypeStruct((rows, dm), f32),
        compiler_params=pltpu.CompilerParams(
            dimension_semantics=("parallel",),
        ),
    )(combined_t.reshape(dm, rows), out_w.astype(bf16), out_b.reshape(1, -1))

    return out.reshape(bsz, t, dm)


# speedup vs baseline: 1.2567x; 1.0222x over previous
from itertools import combinations

import numpy as np
import jax
import jax.numpy as jnp
from jax.experimental import pallas as pl
from jax.experimental.pallas import tpu as pltpu

_D = 1024
_H = 16
_DH = 64
_P = 4
_PD = 6
_NG = 21
_NGP = 24
_DECAY = 0.99
_C = 512
_KC = 512
_RC = 512
_G = 8
_SCALE = _DH ** -0.5
_LN_DECAY = float(np.log(_DECAY))
_DECAY_C = float(_DECAY ** _C)

_PAIRS = list(combinations(range(_P), 2))
_TI, _TJ = np.triu_indices(_PD)


def _np_consts():
    ea = np.zeros((8, 8), np.float32)
    eb = np.zeros((8, 8), np.float32)
    ec = np.zeros((8, 8), np.float32)
    ed = np.zeros((8, 8), np.float32)
    for kk, (i, j) in enumerate(_PAIRS):
        ea[kk, i] = 1.0
        eb[kk, 4 + j] = 1.0
        ec[kk, j] = 1.0
        ed[kk, 4 + i] = 1.0
    eti = np.zeros((_NGP, 8), np.float32)
    etj = np.zeros((_NGP, 8), np.float32)
    for kk in range(_NG):
        eti[kk, _TI[kk]] = 1.0
        etj[kk, _TJ[kk]] = 1.0
    eye = np.eye(_G, dtype=np.float32)
    ea16 = np.kron(eye, ea)
    eb16 = np.kron(eye, eb)
    ec16 = np.kron(eye, ec)
    ed16 = np.kron(eye, ed)
    seg16 = np.kron(eye, np.ones((8, 8), np.float32))
    eti16 = np.kron(eye, eti)
    etj16 = np.kron(eye, etj)
    i = np.arange(_C)[:, None]
    s = np.arange(_C)[None, :]
    lmat = np.where(s < i, _DECAY ** np.maximum(i - 1 - s, 0), 0.0).astype(np.float32)
    lt = np.ascontiguousarray(lmat.T)
    dvec = (_DECAY ** (_C - 1 - np.arange(_C))).astype(np.float32).reshape(1, _C)
    dpow = (_DECAY ** np.arange(_C)).astype(np.float32).reshape(1, _C)
    return ea16, eb16, ec16, ed16, seg16, eti16, etj16, lt, dvec, dpow


_EA, _EB, _EC, _ED, _SEG, _ETI, _ETJ, _LT, _DVEC, _DPOW = _np_consts()


def _proj_kernel(x_ref, qw_ref, qb_ref, w1_ref, w2_ref, gw_ref, gb_ref,
                 qkv_ref, p1_ref, p2_ref, gl_ref):
    xb = x_ref[...]
    cdims = (((0,), (0,)), ((), ()))
    qkv = jax.lax.dot_general(qw_ref[...], xb, cdims,
                              preferred_element_type=jnp.float32) + qb_ref[...]
    qkv_ref[...] = qkv.astype(jnp.bfloat16)
    p1_ref[...] = jax.lax.dot_general(w1_ref[...], xb, cdims,
                                      preferred_element_type=jnp.float32)
    p2_ref[...] = jax.lax.dot_general(w2_ref[...], xb, cdims,
                                      preferred_element_type=jnp.float32)
    gl_ref[...] = jax.lax.dot_general(gw_ref[...], xb, cdims,
                                      preferred_element_type=jnp.float32) + gb_ref[...]


def _attn_gram_kernel(q_ref, k_ref, v_ref, pw_ref, gl_ref, lt_ref, dv_ref,
                      dp_ref, ea_ref, eb_ref, ec_ref, ed_ref, seg_ref,
                      eti_ref, etj_ref, m1_ref, m1b_ref, m2_ref, m2b_ref,
                      out_ref, s_ref):
    qc = pl.program_id(1)
    t0 = qc * _C
    f32 = jnp.float32
    bf16 = jnp.bfloat16

    @pl.when(qc == 0)
    def _():
        s_ref[...] = jnp.zeros((_G * _NGP, 1), f32)

    cdA = (((0,), (0,)), ((), ()))
    cdS = (((1,), (0,)), ((), ()))
    ones_row = jnp.ones((8, _KC), bf16)

    ki = jax.lax.broadcasted_iota(jnp.int32, (_KC, _C), 0)
    qi = jax.lax.broadcasted_iota(jnp.int32, (_KC, _C), 1)
    diag_masks = [ki + moff > qi for moff in range(0, _C, _KC)]

    def one_head_chunk(g, off, m, acc, masked, mi=0):
        kc = k_ref[g, :, pl.ds(off, _KC)]
        st = jax.lax.dot_general(kc, q_ref[g], cdA,
                                 preferred_element_type=f32) * _SCALE
        if masked:
            st = jnp.where(diag_masks[mi], -1e30, st)
        m_new = jnp.maximum(m, jnp.max(st, axis=0, keepdims=True))
        alpha = jnp.exp(m - m_new)
        p = jnp.exp(st - m_new)
        va = jnp.concatenate([v_ref[g, :, pl.ds(off, _KC)], ones_row], axis=0)
        acc_new = acc * alpha + jax.lax.dot_general(
            va, p.astype(bf16), cdS, preferred_element_type=f32)
        return m_new, acc_new

    def body(j, carry):
        off = pl.multiple_of(j * _KC, _KC)
        return tuple(one_head_chunk(g, off, *carry[g], masked=False)
                     for g in range(_G))

    init = tuple((jnp.full((1, _C), -1e30, f32),
                  jnp.zeros((_DH + 8, _C), f32)) for _ in range(_G))
    carry = jax.lax.fori_loop(0, (_C // _KC) * qc, body, init)
    seqs = []
    for g in range(_G):
        m, acc = carry[g]
        for dj in range(_C // _KC):
            m, acc = one_head_chunk(g, t0 + dj * _KC, m, acc,
                                    masked=True, mi=dj)
        seqs.append(acc[0:_DH] / acc[_DH:_DH + 1])

    pwa = pw_ref[...]
    a = jnp.dot(ea_ref[...], pwa, preferred_element_type=f32)
    b = jnp.dot(eb_ref[...], pwa, preferred_element_type=f32)
    c = jnp.dot(ec_ref[...], pwa, preferred_element_type=f32)
    d = jnp.dot(ed_ref[...], pwa, preferred_element_type=f32)
    parts = a * b - c * d
    s2 = jnp.dot(seg_ref[...], (parts * parts).astype(bf16),
                 preferred_element_type=f32)
    nr = jnp.maximum(jnp.sqrt(s2), 1e-12)
    wl = (parts / nr).astype(bf16)
    u = jnp.dot(eti_ref[...], wl, preferred_element_type=f32)
    v = jnp.dot(etj_ref[...], wl, preferred_element_type=f32)
    o = u * v

    carry_s = s_ref[...]
    gf = dp_ref[...] * carry_s + jnp.dot(o.astype(bf16), lt_ref[...],
                                         preferred_element_type=f32)
    s_ref[...] = _DECAY_C * carry_s + jnp.sum(o * dv_ref[...], axis=1, keepdims=True)

    pre = jnp.dot(m1_ref[...], gf.astype(bf16),
                  preferred_element_type=f32) + m1b_ref[...]
    h1 = 0.5 * pre * (1.0 + jax.lax.erf(pre * 0.7071067811865476))
    mem = jnp.dot(m2_ref[...], h1.astype(bf16),
                  preferred_element_type=f32) + m2b_ref[...]

    for g in range(_G):
        gate = jax.nn.sigmoid(gl_ref[0, g:g + 1, :])
        out_ref[g] = (seqs[g] + gate * mem[g * _DH:(g + 1) * _DH]).astype(bf16)


def _out_kernel(c_ref, w_ref, b_ref, o_ref):
    o_ref[...] = jax.lax.dot_general(
        c_ref[...], w_ref[...], (((0,), (0,)), ((), ())),
        preferred_element_type=jnp.float32) + b_ref[...]


def kernel(x, qkv_w, qkv_b, w1_w, w2_w, mlp1_w, mlp1_b, mlp2_w, mlp2_b,
           gate_w, gate_b, out_w, out_b):
    bsz, t, dm = x.shape
    f32 = jnp.float32
    bf16 = jnp.bfloat16
    rows = bsz * t
    ngrid = rows // _RC
    nq = t // _C
    hh = _H

    xt = jnp.transpose(x.reshape(rows, dm).astype(bf16))

    qkvt, p1t, p2t, glt = pl.pallas_call(
        _proj_kernel,
        grid=(ngrid,),
        in_specs=[
            pl.BlockSpec((dm, _RC), lambda i: (0, i)),
            pl.BlockSpec((dm, 3 * dm), lambda i: (0, 0)),
            pl.BlockSpec((3 * dm, 1), lambda i: (0, 0)),
            pl.BlockSpec((dm, _H * _P), lambda i: (0, 0)),
            pl.BlockSpec((dm, _H * _P), lambda i: (0, 0)),
            pl.BlockSpec((dm, _H), lambda i: (0, 0)),
            pl.BlockSpec((_H, 1), lambda i: (0, 0)),
        ],
        out_specs=[
            pl.BlockSpec((3 * dm, _RC), lambda i: (0, i)),
            pl.BlockSpec((_H * _P, _RC), lambda i: (0, i)),
            pl.BlockSpec((_H * _P, _RC), lambda i: (0, i)),
            pl.BlockSpec((_H, _RC), lambda i: (0, i)),
        ],
        out_shape=[
            jax.ShapeDtypeStruct((3 * dm, rows), bf16),
            jax.ShapeDtypeStruct((_H * _P, rows), f32),
            jax.ShapeDtypeStruct((_H * _P, rows), f32),
            jax.ShapeDtypeStruct((_H, rows), f32),
        ],
        compiler_params=pltpu.CompilerParams(
            dimension_semantics=("parallel",),
        ),
    )(xt, qkv_w.astype(bf16), qkv_b.reshape(-1, 1), w1_w.astype(bf16),
      w2_w.astype(bf16), gate_w.astype(bf16), gate_b.reshape(-1, 1))

    qkvh = qkvt.reshape(3 * _H, _DH, rows)
    p1b = p1t.reshape(_H, _P, bsz, t)
    p1s = jnp.concatenate([jnp.zeros((_H, _P, bsz, 1), f32), p1b[..., :-1]], axis=3)
    p2b = p2t.reshape(_H, _P, bsz, t)
    pwt = jnp.concatenate([p1s, p2b], axis=1).reshape(_H, 8, rows)

    m1tp = jnp.concatenate([mlp1_w.T, jnp.zeros((_DH, _NGP - _NG), f32)], axis=1)

    hgn = _H // _G
    combined_t = pl.pallas_call(
        _attn_gram_kernel,
        grid=(bsz * hgn, nq),
        in_specs=[
            pl.BlockSpec((_G, _DH, _C), lambda bh, qc: (bh % hgn, 0, (bh // hgn) * nq + qc)),
            pl.BlockSpec((_G, _DH, t), lambda bh, qc: (hgn + bh % hgn, 0, bh // hgn)),
            pl.BlockSpec((_G, _DH, t), lambda bh, qc: (2 * hgn + bh % hgn, 0, bh // hgn)),
            pl.BlockSpec((_G * 8, _C), lambda bh, qc: (bh % hgn, (bh // hgn) * nq + qc)),
            pl.BlockSpec((1, _G, _C), lambda bh, qc: (bh % hgn, 0, (bh // hgn) * nq + qc)),
            pl.BlockSpec((_C, _C), lambda bh, qc: (0, 0)),
            pl.BlockSpec((1, _C), lambda bh, qc: (0, 0)),
            pl.BlockSpec((1, _C), lambda bh, qc: (0, 0)),
            pl.BlockSpec((_G * 8, _G * 8), lambda bh, qc: (0, 0)),
            pl.BlockSpec((_G * 8, _G * 8), lambda bh, qc: (0, 0)),
            pl.BlockSpec((_G * 8, _G * 8), lambda bh, qc: (0, 0)),
            pl.BlockSpec((_G * 8, _G * 8), lambda bh, qc: (0, 0)),
            pl.BlockSpec((_G * 8, _G * 8), lambda bh, qc: (0, 0)),
            pl.BlockSpec((_G * _NGP, _G * 8), lambda bh, qc: (0, 0)),
            pl.BlockSpec((_G * _NGP, _G * 8), lambda bh, qc: (0, 0)),
            pl.BlockSpec((_G * _DH, _G * _NGP), lambda bh, qc: (0, 0)),
            pl.BlockSpec((_G * _DH, 1), lambda bh, qc: (0, 0)),
            pl.BlockSpec((_G * _DH, _G * _DH), lambda bh, qc: (0, 0)),
            pl.BlockSpec((_G * _DH, 1), lambda bh, qc: (0, 0)),
        ],
        out_specs=pl.BlockSpec((_G, _DH, _C), lambda bh, qc: (bh % hgn, 0, (bh // hgn) * nq + qc)),
        out_shape=jax.ShapeDtypeStruct((_H, _DH, rows), bf16),
        scratch_shapes=[pltpu.VMEM((_G * _NGP, 1), f32)],
        compiler_params=pltpu.CompilerParams(
            dimension_semantics=("parallel", "arbitrary"),
        ),
    )(qkvh, qkvh, qkvh, pwt.reshape(_H * 8, rows).astype(bf16),
      glt.reshape(hgn, _G, rows),
      jnp.asarray(_LT).astype(bf16), jnp.asarray(_DVEC), jnp.asarray(_DPOW),
      jnp.asarray(_EA).astype(bf16), jnp.asarray(_EB).astype(bf16),
      jnp.asarray(_EC).astype(bf16), jnp.asarray(_ED).astype(bf16),
      jnp.asarray(_SEG).astype(bf16),
      jnp.asarray(_ETI).astype(bf16), jnp.asarray(_ETJ).astype(bf16),
      jnp.kron(jnp.eye(_G, dtype=f32), m1tp).astype(bf16),
      jnp.tile(mlp1_b.reshape(-1, 1), (_G, 1)),
      jnp.kron(jnp.eye(_G, dtype=f32), mlp2_w.T).astype(bf16),
      jnp.tile(mlp2_b.reshape(-1, 1), (_G, 1)))

    out = pl.pallas_call(
        _out_kernel,
        grid=(ngrid,),
        in_specs=[
            pl.BlockSpec((dm, _RC), lambda i: (0, i)),
            pl.BlockSpec((dm, dm), lambda i: (0, 0)),
            pl.BlockSpec((1, dm), lambda i: (0, 0)),
        ],
        out_specs=pl.BlockSpec((_RC, dm), lambda i: (i, 0)),
        out_shape=jax.ShapeDtypeStruct((rows, dm), f32),
        compiler_params=pltpu.CompilerParams(
            dimension_semantics=("parallel",),
        ),
    )(combined_t.reshape(dm, rows), out_w.astype(bf16), out_b.reshape(1, -1))

    return out.reshape(bsz, t, dm)
